# baseline (speedup 1.0000x reference)
import jax
import jax.numpy as jnp
from jax import lax
from jax.experimental import pallas as pl
from jax.experimental.pallas import tpu as pltpu
from jax.experimental.pallas import tpu_sc as plsc

_N = 100000
_G = 256
_H = 64
_NC, _NS = 2, 16
_NPAD = 100352
_BN = 2048
_GRID = _NPAD // _BN
_E = 3200000
_KB = 6
_EROWS = 25344
_EPAD = _EROWS * 128
_RPW_A = _EROWS // (_NC * _NS)
_NB_A = _RPW_A // _KB
_RPW_C = _EROWS // _NS
_NB_C = _RPW_C // _KB
_ZR = _NPAD // _NS

_mesh = plsc.VectorSubcoreMesh(core_axis_name="c", subcore_axis_name="s")


def _sc_pipeline(nb, base, src_hbm, dst_hbm, tab_hbm, table, zeros, src_v,
                 dst_v, rows_v, sem_g, sem_s0, sem_s1, sem_i):
    pltpu.async_copy(src_hbm.at[pl.ds(base, _KB)], src_v.at[0], sem_i)
    pltpu.async_copy(dst_hbm.at[pl.ds(base, _KB)], dst_v.at[0], sem_i)
    zrows = zeros.at[pl.ds(0, _KB * 128)]

    def body(k, carry):
        for p, sem_s in ((0, sem_s0), (1, sem_s1)):
            i = 2 * k + p
            s3 = lax.rem(i, 3)

            @pl.when(k >= 1)
            def _():
                pltpu.make_async_copy(zrows, rows_v.at[p], sem_s).wait()

            pltpu.make_async_copy(src_hbm.at[pl.ds(base, _KB)],
                                  src_v.at[s3], sem_i).wait()
            pltpu.make_async_copy(dst_hbm.at[pl.ds(base, _KB)],
                                  dst_v.at[s3], sem_i).wait()

            @pl.when(i + 1 < nb)
            def _():
                r1 = base + (i + 1) * _KB
                nx = lax.rem(i + 1, 3)
                pltpu.async_copy(src_hbm.at[pl.ds(r1, _KB)], src_v.at[nx],
                                 sem_i)
                pltpu.async_copy(dst_hbm.at[pl.ds(r1, _KB)], dst_v.at[nx],
                                 sem_i)

            for j in range(_KB):
                pltpu.async_copy(tab_hbm.at[src_v.at[s3, j]],
                                 rows_v.at[p, pl.ds(j * 128, 128)], sem_g)
            pltpu.make_async_copy(zrows, rows_v.at[p], sem_g).wait()
            for j in range(_KB):
                pltpu.async_copy(rows_v.at[p, pl.ds(j * 128, 128)],
                                 table.at[dst_v.at[s3, j]], sem_s, add=True)
        return carry

    lax.fori_loop(0, nb // 2, body, 0)
    pltpu.make_async_copy(zrows, rows_v.at[0], sem_s0).wait()
    pltpu.make_async_copy(zrows, rows_v.at[1], sem_s1).wait()


def _agg_a_body(h0, src_hbm, dst_hbm, zeros, p0, p1, table, src_v, dst_v,
                rows_v, sem_g, sem_s0, sem_s1, sem_i):
    cid = lax.axis_index("c")
    sid = lax.axis_index("s")
    z0 = sid * _ZR
    pltpu.sync_copy(zeros.at[pl.ds(z0, _ZR)], table.at[pl.ds(z0, _ZR)])
    plsc.subcore_barrier()
    base = (cid * _NS + sid) * _RPW_A
    _sc_pipeline(_NB_A, base, src_hbm, dst_hbm, h0, table, zeros, src_v,
                 dst_v, rows_v, sem_g, sem_s0, sem_s1, sem_i)
    plsc.subcore_barrier()

    @pl.when(cid == 0)
    def _():
        pltpu.sync_copy(table.at[pl.ds(z0, _ZR)], p0.at[pl.ds(z0, _ZR)])

    @pl.when(cid == 1)
    def _():
        pltpu.sync_copy(table.at[pl.ds(z0, _ZR)], p1.at[pl.ds(z0, _ZR)])


def _agg_c_body(hc0, hc1, hc2, hc3, src_hbm, dst_hbm, zeros, a0, a1, a2, a3,
                table, src_v, dst_v, rows_v, sem_g, sem_s0, sem_s1, sem_i):
    cid = lax.axis_index("c")
    sid = lax.axis_index("s")
    z0 = sid * _ZR
    base = sid * _RPW_C

    def one_pass(tab_hbm, out_hbm):
        pltpu.sync_copy(zeros.at[pl.ds(z0, _ZR)], table.at[pl.ds(z0, _ZR)])
        plsc.subcore_barrier()
        _sc_pipeline(_NB_C, base, src_hbm, dst_hbm, tab_hbm, table, zeros,
                     src_v, dst_v, rows_v, sem_g, sem_s0, sem_s1, sem_i)
        plsc.subcore_barrier()
        pltpu.sync_copy(table.at[pl.ds(z0, _ZR)], out_hbm.at[pl.ds(z0, _ZR)])

    @pl.when(cid == 0)
    def _():
        one_pass(hc0, a0)
        one_pass(hc1, a1)

    @pl.when(cid == 1)
    def _():
        one_pass(hc2, a2)
        one_pass(hc3, a3)


_sc_params = pltpu.CompilerParams(use_tc_tiling_on_sc=False)

_agg_a = pl.kernel(
    _agg_a_body,
    out_type=[jax.ShapeDtypeStruct((_NPAD, 16), jnp.float32)] * 2,
    mesh=_mesh,
    compiler_params=_sc_params,
    scratch_types=[
        pltpu.VMEM_SHARED((_NPAD, 16), jnp.float32),
        pltpu.VMEM((3, _KB, 128), jnp.int32),
        pltpu.VMEM((3, _KB, 128), jnp.int32),
        pltpu.VMEM((2, _KB * 128, 16), jnp.float32),
        pltpu.SemaphoreType.DMA,
        pltpu.SemaphoreType.DMA,
        pltpu.SemaphoreType.DMA,
        pltpu.SemaphoreType.DMA,
    ],
)

_agg_c = pl.kernel(
    _agg_c_body,
    out_type=[jax.ShapeDtypeStruct((_NPAD, 16), jnp.float32)] * 4,
    mesh=_mesh,
    compiler_params=_sc_params,
    scratch_types=[
        pltpu.VMEM_SHARED((_NPAD, 16), jnp.float32),
        pltpu.VMEM((3, _KB, 128), jnp.int32),
        pltpu.VMEM((3, _KB, 128), jnp.int32),
        pltpu.VMEM((2, _KB * 128, 16), jnp.float32),
        pltpu.SemaphoreType.DMA,
        pltpu.SemaphoreType.DMA,
        pltpu.SemaphoreType.DMA,
        pltpu.SemaphoreType.DMA,
    ],
)


def _mlp_a_body(h0, p0, p1, w1, b1, w2, b2, o0, o1, o2, o3):
    hin = h0[...] + p0[...] + p1[...]
    z = jnp.maximum(hin @ w1[...] + b1[...], 0.0)
    h1 = jnp.maximum(jnp.maximum(z @ w2[...] + b2[...], 0.0), 0.0)
    o0[...] = h1[:, 0:16]
    o1[...] = h1[:, 16:32]
    o2[...] = h1[:, 32:48]
    o3[...] = h1[:, 48:64]


def _mlp_b_body(hc0, hc1, hc2, hc3, a0, a1, a2, a3, bat, w1, b1, w2, b2, wl,
                bl, out, acc):
    i = pl.program_id(0)

    @pl.when(i == 0)
    def _():
        acc[...] = jnp.zeros_like(acc)

    hin = jnp.concatenate(
        [hc0[...] + a0[...], hc1[...] + a1[...], hc2[...] + a2[...],
         hc3[...] + a3[...]], axis=1)
    z = jnp.maximum(hin @ w1[...] + b1[...], 0.0)
    h2 = jnp.maximum(z @ w2[...] + b2[...], 0.0)
    onehot = (bat[...] == lax.broadcasted_iota(jnp.int32, (_BN, _G), 1)
              ).astype(jnp.float32)
    ext = jnp.concatenate([h2, jnp.ones((_BN, _H), jnp.float32)], axis=1)
    acc[...] += lax.dot_general(onehot, ext, (((0,), (0,)), ((), ())))

    @pl.when(i == _GRID - 1)
    def _():
        s = acc[...]
        mean = s[:, :_H] / jnp.maximum(s[:, _H:_H + 1], 1.0)
        out[...] = mean @ wl[...] + bl[0, 0]


_row_spec = pl.BlockSpec((_BN, 16), lambda i: (i, 0))


def _full(shape):
    return pl.BlockSpec(shape, lambda i: tuple(0 for _ in shape))


_mlp_a = pl.pallas_call(
    _mlp_a_body,
    grid=(_GRID,),
    in_specs=[_row_spec, _row_spec, _row_spec,
              _full((16, _H)), _full((1, _H)), _full((_H, _H)),
              _full((1, _H))],
    out_specs=[_row_spec] * 4,
    out_shape=[jax.ShapeDtypeStruct((_NPAD, 16), jnp.float32)] * 4,
)

_mlp_b = pl.pallas_call(
    _mlp_b_body,
    grid=(_GRID,),
    in_specs=[_row_spec] * 8 + [
        pl.BlockSpec((_BN, 1), lambda i: (i, 0)),
        _full((_H, _H)), _full((1, _H)), _full((_H, _H)), _full((1, _H)),
        _full((_H, 1)), _full((1, 1))],
    out_specs=_full((_G, 1)),
    out_shape=jax.ShapeDtypeStruct((_G, 1), jnp.float32),
    scratch_shapes=[pltpu.VMEM((_G, 2 * _H), jnp.float32)],
)


def kernel(x, pos, edge_index, batch, W1a, b1a, W2a, b2a, W1b, b1b, W2b, b2b,
           Wl, bl):
    h0 = jnp.concatenate([x, pos], axis=1)
    h0 = jnp.pad(h0, ((0, _NPAD - _N), (0, 16 - h0.shape[1])))
    src = jnp.pad(edge_index[0], (0, _EPAD - _E),
                  constant_values=_N).reshape(_EROWS, 128)
    dst = jnp.pad(edge_index[1], (0, _EPAD - _E),
                  constant_values=_N).reshape(_EROWS, 128)
    zeros = jnp.zeros((_NPAD, 16), jnp.float32)
    w1a = jnp.pad(W1a, ((0, 16 - W1a.shape[0]), (0, 0)))
    bat = jnp.pad(batch, (0, _NPAD - _N), constant_values=_G).reshape(_NPAD, 1)

    p0, p1 = _agg_a(h0, src, dst, zeros)
    hc = _mlp_a(h0, p0, p1, w1a, b1a.reshape(1, _H), W2a, b2a.reshape(1, _H))
    ac = _agg_c(hc[0], hc[1], hc[2], hc[3], src, dst, zeros)
    out = _mlp_b(hc[0], hc[1], hc[2], hc[3], ac[0], ac[1], ac[2], ac[3], bat,
                 W1b, b1b.reshape(1, _H), W2b, b2b.reshape(1, _H), Wl,
                 bl.reshape(1, 1))
    return out


# speedup vs baseline: 1.0033x; 1.0033x over previous
import jax
import jax.numpy as jnp
from jax import lax
from jax.experimental import pallas as pl
from jax.experimental.pallas import tpu as pltpu
from jax.experimental.pallas import tpu_sc as plsc

_N = 100000
_G = 256
_H = 64
_NC, _NS = 2, 16
_NPAD = 100352
_BN = 2048
_GRID = _NPAD // _BN
_E = 3200000
_KB = 6
_EROWS = 25344
_EPAD = _EROWS * 128
_RPW_A = _EROWS // (_NC * _NS)
_NB_A = _RPW_A // _KB
_RPW_C = _EROWS // _NS
_NB_C = _RPW_C // _KB
_ZR = _NPAD // _NS

_mesh = plsc.VectorSubcoreMesh(core_axis_name="c", subcore_axis_name="s")


def _sc_pipeline(nb, base, src_hbm, dst_hbm, tab_hbm, table, zdrain, src_v,
                 dst_v, rows_v, sem_g, sem_s0, sem_s1, sem_i):
    eb = base * 128
    pltpu.async_copy(src_hbm.at[pl.ds(eb, _KB * 128)], src_v.at[0], sem_i)
    pltpu.async_copy(dst_hbm.at[pl.ds(eb, _KB * 128)], dst_v.at[0], sem_i)
    zrows = zdrain.at[pl.ds(0, _KB * 128)]

    def body(k, carry):
        for p, sem_s in ((0, sem_s0), (1, sem_s1)):
            i = 2 * k + p
            s3 = lax.rem(i, 3)

            @pl.when(k >= 1)
            def _():
                pltpu.make_async_copy(zrows, rows_v.at[p], sem_s).wait()

            pltpu.make_async_copy(src_hbm.at[pl.ds(eb, _KB * 128)],
                                  src_v.at[s3], sem_i).wait()
            pltpu.make_async_copy(dst_hbm.at[pl.ds(eb, _KB * 128)],
                                  dst_v.at[s3], sem_i).wait()

            @pl.when(i + 1 < nb)
            def _():
                r1 = eb + (i + 1) * _KB * 128
                nx = lax.rem(i + 1, 3)
                pltpu.async_copy(src_hbm.at[pl.ds(r1, _KB * 128)],
                                 src_v.at[nx], sem_i)
                pltpu.async_copy(dst_hbm.at[pl.ds(r1, _KB * 128)],
                                 dst_v.at[nx], sem_i)

            pltpu.async_copy(tab_hbm.at[src_v.at[s3]], rows_v.at[p],
                             sem_g).wait()
            pltpu.async_copy(rows_v.at[p], table.at[dst_v.at[s3]], sem_s,
                             add=True)
        return carry

    lax.fori_loop(0, nb // 2, body, 0)
    pltpu.make_async_copy(zrows, rows_v.at[0], sem_s0).wait()
    pltpu.make_async_copy(zrows, rows_v.at[1], sem_s1).wait()


def _agg_a_body(h0, src_hbm, dst_hbm, zeros, zdrain, p0, p1, table, src_v,
                dst_v, rows_v, sem_g, sem_s0, sem_s1, sem_i):
    cid = lax.axis_index("c")
    sid = lax.axis_index("s")
    z0 = sid * _ZR
    pltpu.sync_copy(zeros.at[pl.ds(z0, _ZR)], table.at[pl.ds(z0, _ZR)])
    plsc.subcore_barrier()
    base = (cid * _NS + sid) * _RPW_A
    _sc_pipeline(_NB_A, base, src_hbm, dst_hbm, h0, table, zdrain, src_v,
                 dst_v, rows_v, sem_g, sem_s0, sem_s1, sem_i)
    plsc.subcore_barrier()

    @pl.when(cid == 0)
    def _():
        pltpu.sync_copy(table.at[pl.ds(z0, _ZR)], p0.at[pl.ds(z0, _ZR)])

    @pl.when(cid == 1)
    def _():
        pltpu.sync_copy(table.at[pl.ds(z0, _ZR)], p1.at[pl.ds(z0, _ZR)])


def _agg_c_body(hc0, hc1, hc2, hc3, src_hbm, dst_hbm, zeros, zdrain, a0, a1,
                a2, a3, table, src_v, dst_v, rows_v, sem_g, sem_s0, sem_s1,
                sem_i):
    cid = lax.axis_index("c")
    sid = lax.axis_index("s")
    z0 = sid * _ZR
    base = sid * _RPW_C

    def one_pass(tab_hbm, out_hbm):
        pltpu.sync_copy(zeros.at[pl.ds(z0, _ZR)], table.at[pl.ds(z0, _ZR)])
        plsc.subcore_barrier()
        _sc_pipeline(_NB_C, base, src_hbm, dst_hbm, tab_hbm, table, zdrain,
                     src_v, dst_v, rows_v, sem_g, sem_s0, sem_s1, sem_i)
        plsc.subcore_barrier()
        pltpu.sync_copy(table.at[pl.ds(z0, _ZR)], out_hbm.at[pl.ds(z0, _ZR)])

    @pl.when(cid == 0)
    def _():
        one_pass(hc0, a0)
        one_pass(hc1, a1)

    @pl.when(cid == 1)
    def _():
        one_pass(hc2, a2)
        one_pass(hc3, a3)


_sc_params = pltpu.CompilerParams(use_tc_tiling_on_sc=False)

_agg_a = pl.kernel(
    _agg_a_body,
    out_type=[jax.ShapeDtypeStruct((_NPAD, 16), jnp.float32)] * 2,
    mesh=_mesh,
    compiler_params=_sc_params,
    scratch_types=[
        pltpu.VMEM_SHARED((_NPAD, 16), jnp.float32),
        pltpu.VMEM((3, _KB * 128), jnp.int32),
        pltpu.VMEM((3, _KB * 128), jnp.int32),
        pltpu.VMEM((2, _KB * 128, 16), jnp.float32),
        pltpu.SemaphoreType.DMA,
        pltpu.SemaphoreType.DMA,
        pltpu.SemaphoreType.DMA,
        pltpu.SemaphoreType.DMA,
    ],
)

_agg_c = pl.kernel(
    _agg_c_body,
    out_type=[jax.ShapeDtypeStruct((_NPAD, 16), jnp.float32)] * 4,
    mesh=_mesh,
    compiler_params=_sc_params,
    scratch_types=[
        pltpu.VMEM_SHARED((_NPAD, 16), jnp.float32),
        pltpu.VMEM((3, _KB * 128), jnp.int32),
        pltpu.VMEM((3, _KB * 128), jnp.int32),
        pltpu.VMEM((2, _KB * 128, 16), jnp.float32),
        pltpu.SemaphoreType.DMA,
        pltpu.SemaphoreType.DMA,
        pltpu.SemaphoreType.DMA,
        pltpu.SemaphoreType.DMA,
    ],
)


def _mlp_a_body(h0, p0, p1, w1, b1, w2, b2, o0, o1, o2, o3):
    hin = h0[...] + p0[...] + p1[...]
    z = jnp.maximum(hin @ w1[...] + b1[...], 0.0)
    h1 = jnp.maximum(jnp.maximum(z @ w2[...] + b2[...], 0.0), 0.0)
    o0[...] = h1[:, 0:16]
    o1[...] = h1[:, 16:32]
    o2[...] = h1[:, 32:48]
    o3[...] = h1[:, 48:64]


def _mlp_b_body(hc0, hc1, hc2, hc3, a0, a1, a2, a3, bat, w1, b1, w2, b2, wl,
                bl, out, acc):
    i = pl.program_id(0)

    @pl.when(i == 0)
    def _():
        acc[...] = jnp.zeros_like(acc)

    hin = jnp.concatenate(
        [hc0[...] + a0[...], hc1[...] + a1[...], hc2[...] + a2[...],
         hc3[...] + a3[...]], axis=1)
    z = jnp.maximum(hin @ w1[...] + b1[...], 0.0)
    h2 = jnp.maximum(z @ w2[...] + b2[...], 0.0)
    onehot = (bat[...] == lax.broadcasted_iota(jnp.int32, (_BN, _G), 1)
              ).astype(jnp.float32)
    ext = jnp.concatenate([h2, jnp.ones((_BN, _H), jnp.float32)], axis=1)
    acc[...] += lax.dot_general(onehot, ext, (((0,), (0,)), ((), ())))

    @pl.when(i == _GRID - 1)
    def _():
        s = acc[...]
        mean = s[:, :_H] / jnp.maximum(s[:, _H:_H + 1], 1.0)
        out[...] = mean @ wl[...] + bl[0, 0]


_row_spec = pl.BlockSpec((_BN, 16), lambda i: (i, 0))


def _full(shape):
    return pl.BlockSpec(shape, lambda i: tuple(0 for _ in shape))


_mlp_a = pl.pallas_call(
    _mlp_a_body,
    grid=(_GRID,),
    in_specs=[_row_spec, _row_spec, _row_spec,
              _full((16, _H)), _full((1, _H)), _full((_H, _H)),
              _full((1, _H))],
    out_specs=[_row_spec] * 4,
    out_shape=[jax.ShapeDtypeStruct((_NPAD, 16), jnp.float32)] * 4,
)

_mlp_b = pl.pallas_call(
    _mlp_b_body,
    grid=(_GRID,),
    in_specs=[_row_spec] * 8 + [
        pl.BlockSpec((_BN, 1), lambda i: (i, 0)),
        _full((_H, _H)), _full((1, _H)), _full((_H, _H)), _full((1, _H)),
        _full((_H, 1)), _full((1, 1))],
    out_specs=_full((_G, 1)),
    out_shape=jax.ShapeDtypeStruct((_G, 1), jnp.float32),
    scratch_shapes=[pltpu.VMEM((_G, 2 * _H), jnp.float32)],
)


def kernel(x, pos, edge_index, batch, W1a, b1a, W2a, b2a, W1b, b1b, W2b, b2b,
           Wl, bl):
    h0 = jnp.concatenate([x, pos], axis=1)
    h0 = jnp.pad(h0, ((0, _NPAD - _N), (0, 16 - h0.shape[1])))
    src = jnp.pad(edge_index[0], (0, _EPAD - _E), constant_values=_N)
    dst = jnp.pad(edge_index[1], (0, _EPAD - _E), constant_values=_N)
    zeros = jnp.zeros((_NPAD, 16), jnp.float32)
    w1a = jnp.pad(W1a, ((0, 16 - W1a.shape[0]), (0, 0)))
    bat = jnp.pad(batch, (0, _NPAD - _N), constant_values=_G).reshape(_NPAD, 1)

    p0, p1 = _agg_a(h0, src, dst, zeros, zeros)
    hc = _mlp_a(h0, p0, p1, w1a, b1a.reshape(1, _H), W2a, b2a.reshape(1, _H))
    ac = _agg_c(hc[0], hc[1], hc[2], hc[3], src, dst, zeros, zeros)
    out = _mlp_b(hc[0], hc[1], hc[2], hc[3], ac[0], ac[1], ac[2], ac[3], bat,
                 W1b, b1b.reshape(1, _H), W2b, b2b.reshape(1, _H), Wl,
                 bl.reshape(1, 1))
    return out


# speedup vs baseline: 1.3838x; 1.3792x over previous
import jax
import jax.numpy as jnp
from jax import lax
from jax.experimental import pallas as pl
from jax.experimental.pallas import tpu as pltpu
from jax.experimental.pallas import tpu_sc as plsc

_N = 100000
_G = 256
_H = 64
_NC, _NS = 2, 16
_NPAD = 100352
_BN = 2048
_GRID = _NPAD // _BN
_E = 3200000
_KB = 6
_EB = _KB * 128
_EPW_A = _E // (_NC * _NS)
_NB_A = _EPW_A // _EB // 2 * 2
_TAIL_A = _EPW_A - _NB_A * _EB
_EPW_C = _E // _NS
_NB_C = _EPW_C // _EB // 2 * 2
_TAIL_C = _EPW_C - _NB_C * _EB
_ZR = _NPAD // _NS

_mesh = plsc.VectorSubcoreMesh(core_axis_name="c", subcore_axis_name="s")


def _sc_pipeline(nb, tail, base, src_hbm, dst_hbm, tab_hbm, table, zdrain,
                 src_v, dst_v, rows_v, st_v, dt_v, sem_g, sem_s0, sem_s1,
                 sem_i):
    pltpu.async_copy(src_hbm.at[pl.ds(base, _EB)], src_v.at[0], sem_i)
    pltpu.async_copy(dst_hbm.at[pl.ds(base, _EB)], dst_v.at[0], sem_i)
    zrows = zdrain.at[pl.ds(0, _EB)]

    def body(k, carry):
        for p, sem_s in ((0, sem_s0), (1, sem_s1)):
            i = 2 * k + p
            s3 = lax.rem(i, 3)

            @pl.when(k >= 1)
            def _():
                pltpu.make_async_copy(zrows, rows_v.at[p], sem_s).wait()

            pltpu.make_async_copy(src_hbm.at[pl.ds(base, _EB)],
                                  src_v.at[s3], sem_i).wait()
            pltpu.make_async_copy(dst_hbm.at[pl.ds(base, _EB)],
                                  dst_v.at[s3], sem_i).wait()

            @pl.when(i + 1 < nb)
            def _():
                r1 = base + (i + 1) * _EB
                nx = lax.rem(i + 1, 3)
                pltpu.async_copy(src_hbm.at[pl.ds(r1, _EB)], src_v.at[nx],
                                 sem_i)
                pltpu.async_copy(dst_hbm.at[pl.ds(r1, _EB)], dst_v.at[nx],
                                 sem_i)

            pltpu.async_copy(tab_hbm.at[src_v.at[s3]], rows_v.at[p],
                             sem_g).wait()
            pltpu.async_copy(rows_v.at[p], table.at[dst_v.at[s3]], sem_s,
                             add=True)
        return carry

    lax.fori_loop(0, nb // 2, body, 0)
    pltpu.make_async_copy(zrows, rows_v.at[0], sem_s0).wait()
    pltpu.make_async_copy(zrows, rows_v.at[1], sem_s1).wait()

    t0 = base + nb * _EB
    pltpu.sync_copy(src_hbm.at[pl.ds(t0, tail)], st_v)
    pltpu.sync_copy(dst_hbm.at[pl.ds(t0, tail)], dt_v)
    pltpu.async_copy(tab_hbm.at[st_v], rows_v.at[0, pl.ds(0, tail)],
                     sem_g).wait()
    pltpu.sync_copy(rows_v.at[0, pl.ds(0, tail)], table.at[dt_v], add=True)


def _agg_a_body(h0, src_hbm, dst_hbm, zeros, zdrain, p0, p1, table, src_v,
                dst_v, rows_v, st_v, dt_v, sem_g, sem_s0, sem_s1, sem_i):
    cid = lax.axis_index("c")
    sid = lax.axis_index("s")
    z0 = sid * _ZR
    pltpu.sync_copy(zeros.at[pl.ds(z0, _ZR)], table.at[pl.ds(z0, _ZR)])
    plsc.subcore_barrier()
    base = (cid * _NS + sid) * _EPW_A
    _sc_pipeline(_NB_A, _TAIL_A, base, src_hbm, dst_hbm, h0, table, zdrain,
                 src_v, dst_v, rows_v, st_v, dt_v, sem_g, sem_s0, sem_s1,
                 sem_i)
    plsc.subcore_barrier()

    @pl.when(cid == 0)
    def _():
        pltpu.sync_copy(table.at[pl.ds(z0, _ZR)], p0.at[pl.ds(z0, _ZR)])

    @pl.when(cid == 1)
    def _():
        pltpu.sync_copy(table.at[pl.ds(z0, _ZR)], p1.at[pl.ds(z0, _ZR)])


def _agg_c_body(hc0, hc1, hc2, hc3, src_hbm, dst_hbm, zeros, zdrain, a0, a1,
                a2, a3, table, src_v, dst_v, rows_v, st_v, dt_v, sem_g,
                sem_s0, sem_s1, sem_i):
    cid = lax.axis_index("c")
    sid = lax.axis_index("s")
    z0 = sid * _ZR
    base = sid * _EPW_C

    def one_pass(tab_hbm, out_hbm):
        pltpu.sync_copy(zeros.at[pl.ds(z0, _ZR)], table.at[pl.ds(z0, _ZR)])
        plsc.subcore_barrier()
        _sc_pipeline(_NB_C, _TAIL_C, base, src_hbm, dst_hbm, tab_hbm, table,
                     zdrain, src_v, dst_v, rows_v, st_v, dt_v, sem_g, sem_s0,
                     sem_s1, sem_i)
        plsc.subcore_barrier()
        pltpu.sync_copy(table.at[pl.ds(z0, _ZR)], out_hbm.at[pl.ds(z0, _ZR)])

    @pl.when(cid == 0)
    def _():
        one_pass(hc0, a0)
        one_pass(hc1, a1)

    @pl.when(cid == 1)
    def _():
        one_pass(hc2, a2)
        one_pass(hc3, a3)


_sc_params = pltpu.CompilerParams(use_tc_tiling_on_sc=False)

_agg_a = pl.kernel(
    _agg_a_body,
    out_type=[jax.ShapeDtypeStruct((_NPAD, 16), jnp.float32)] * 2,
    mesh=_mesh,
    compiler_params=_sc_params,
    scratch_types=[
        pltpu.VMEM_SHARED((_NPAD, 16), jnp.float32),
        pltpu.VMEM((3, _EB), jnp.int32),
        pltpu.VMEM((3, _EB), jnp.int32),
        pltpu.VMEM((2, _EB, 16), jnp.float32),
        pltpu.VMEM((_TAIL_A,), jnp.int32),
        pltpu.VMEM((_TAIL_A,), jnp.int32),
        pltpu.SemaphoreType.DMA,
        pltpu.SemaphoreType.DMA,
        pltpu.SemaphoreType.DMA,
        pltpu.SemaphoreType.DMA,
    ],
)

_agg_c = pl.kernel(
    _agg_c_body,
    out_type=[jax.ShapeDtypeStruct((_NPAD, 16), jnp.float32)] * 4,
    mesh=_mesh,
    compiler_params=_sc_params,
    scratch_types=[
        pltpu.VMEM_SHARED((_NPAD, 16), jnp.float32),
        pltpu.VMEM((3, _EB), jnp.int32),
        pltpu.VMEM((3, _EB), jnp.int32),
        pltpu.VMEM((2, _EB, 16), jnp.float32),
        pltpu.VMEM((_TAIL_C,), jnp.int32),
        pltpu.VMEM((_TAIL_C,), jnp.int32),
        pltpu.SemaphoreType.DMA,
        pltpu.SemaphoreType.DMA,
        pltpu.SemaphoreType.DMA,
        pltpu.SemaphoreType.DMA,
    ],
)


def _mlp_a_body(h0, p0, p1, w1, b1, w2, b2, o0, o1, o2, o3):
    hin = h0[...] + p0[...] + p1[...]
    z = jnp.maximum(hin @ w1[...] + b1[...], 0.0)
    h1 = jnp.maximum(jnp.maximum(z @ w2[...] + b2[...], 0.0), 0.0)
    o0[...] = h1[:, 0:16]
    o1[...] = h1[:, 16:32]
    o2[...] = h1[:, 32:48]
    o3[...] = h1[:, 48:64]


def _mlp_b_body(hc0, hc1, hc2, hc3, a0, a1, a2, a3, bat, w1, b1, w2, b2, wl,
                bl, out, acc):
    i = pl.program_id(0)

    @pl.when(i == 0)
    def _():
        acc[...] = jnp.zeros_like(acc)

    hin = jnp.concatenate(
        [hc0[...] + a0[...], hc1[...] + a1[...], hc2[...] + a2[...],
         hc3[...] + a3[...]], axis=1)
    z = jnp.maximum(hin @ w1[...] + b1[...], 0.0)
    h2 = jnp.maximum(z @ w2[...] + b2[...], 0.0)
    onehot = (bat[...] == lax.broadcasted_iota(jnp.int32, (_BN, _G), 1)
              ).astype(jnp.float32)
    ext = jnp.concatenate([h2, jnp.ones((_BN, _H), jnp.float32)], axis=1)
    acc[...] += lax.dot_general(onehot, ext, (((0,), (0,)), ((), ())))

    @pl.when(i == _GRID - 1)
    def _():
        s = acc[...]
        mean = s[:, :_H] / jnp.maximum(s[:, _H:_H + 1], 1.0)
        out[...] = mean @ wl[...] + bl[0, 0]


_row_spec = pl.BlockSpec((_BN, 16), lambda i: (i, 0))


def _full(shape):
    return pl.BlockSpec(shape, lambda i: tuple(0 for _ in shape))


_mlp_a = pl.pallas_call(
    _mlp_a_body,
    grid=(_GRID,),
    in_specs=[_row_spec, _row_spec, _row_spec,
              _full((16, _H)), _full((1, _H)), _full((_H, _H)),
              _full((1, _H))],
    out_specs=[_row_spec] * 4,
    out_shape=[jax.ShapeDtypeStruct((_NPAD, 16), jnp.float32)] * 4,
)

_mlp_b = pl.pallas_call(
    _mlp_b_body,
    grid=(_GRID,),
    in_specs=[_row_spec] * 8 + [
        pl.BlockSpec((_BN, 1), lambda i: (i, 0)),
        _full((_H, _H)), _full((1, _H)), _full((_H, _H)), _full((1, _H)),
        _full((_H, 1)), _full((1, 1))],
    out_specs=_full((_G, 1)),
    out_shape=jax.ShapeDtypeStruct((_G, 1), jnp.float32),
    scratch_shapes=[pltpu.VMEM((_G, 2 * _H), jnp.float32)],
)


def kernel(x, pos, edge_index, batch, W1a, b1a, W2a, b2a, W1b, b1b, W2b, b2b,
           Wl, bl):
    h0 = jnp.concatenate([x, pos], axis=1)
    h0 = jnp.pad(h0, ((0, _NPAD - _N), (0, 16 - h0.shape[1])))
    src = edge_index[0]
    dst = edge_index[1]
    zeros = jnp.zeros((_NPAD, 16), jnp.float32)
    w1a = jnp.pad(W1a, ((0, 16 - W1a.shape[0]), (0, 0)))
    bat = jnp.pad(batch, (0, _NPAD - _N), constant_values=_G).reshape(_NPAD, 1)

    p0, p1 = _agg_a(h0, src, dst, zeros, zeros)
    hc = _mlp_a(h0, p0, p1, w1a, b1a.reshape(1, _H), W2a, b2a.reshape(1, _H))
    ac = _agg_c(hc[0], hc[1], hc[2], hc[3], src, dst, zeros, zeros)
    out = _mlp_b(hc[0], hc[1], hc[2], hc[3], ac[0], ac[1], ac[2], ac[3], bat,
                 W1b, b1b.reshape(1, _H), W2b, b2b.reshape(1, _H), Wl,
                 bl.reshape(1, 1))
    return out
